# Initial kernel scaffold; baseline (speedup 1.0000x reference)
import jax
import jax.numpy as jnp
from jax.experimental import pallas as pl


def kernel(h, edge_index, e, snorm_n, W_pre, b_pre, W_post, b_post, gamma, beta):
    raise NotImplementedError("write your pallas kernel here")


# speedup vs baseline: 1.5045x; 1.5045x over previous
import functools

import jax
import jax.numpy as jnp
from jax.experimental import pallas as pl
from jax.experimental.pallas import tpu as pltpu

_EPS = 1e-5
_AVG_D_LOG = 3.5


def _ab_body(h_ref, w_ref, ab_ref):
    ab_ref[...] = jax.lax.dot_general(
        h_ref[...], w_ref[...], (((1,), (0,)), ((), ())),
        preferred_element_type=jnp.float32)


def _c_body(e_ref, w_ref, b_ref, c_ref):
    c_ref[...] = jax.lax.dot_general(
        e_ref[...], w_ref[...], (((1,), (0,)), ((), ())),
        preferred_element_type=jnp.float32) + b_ref[...]


def _seg_body(ei_ref, a_ref, c_ref, s1_ref, s2_ref, mx_ref, mn_ref, deg_ref,
              *, eb):
    i = pl.program_id(0)

    @pl.when(i == 0)
    def _():
        s1_ref[...] = jnp.zeros_like(s1_ref)
        s2_ref[...] = jnp.zeros_like(s2_ref)
        mx_ref[...] = jnp.full_like(mx_ref, -jnp.inf)
        mn_ref[...] = jnp.full_like(mn_ref, jnp.inf)
        deg_ref[...] = jnp.zeros_like(deg_ref)

    def body(k, carry):
        s = ei_ref[0, 0, k]
        d = ei_ref[0, 1, k]
        u = a_ref[pl.ds(s, 1), :] + c_ref[pl.ds(k, 1), :]
        s1_ref[pl.ds(d, 1), :] += u
        s2_ref[pl.ds(d, 1), :] += u * u
        mx_ref[pl.ds(d, 1), :] = jnp.maximum(mx_ref[pl.ds(d, 1), :], u)
        mn_ref[pl.ds(d, 1), :] = jnp.minimum(mn_ref[pl.ds(d, 1), :], u)
        deg_ref[pl.ds(d, 1), :] += 1.0
        return carry

    jax.lax.fori_loop(0, eb, body, 0)


def _post_body(h_ref, ab_ref, s1_ref, s2_ref, mx_ref, mn_ref, deg_ref,
               sn_ref, wp_ref, bp_ref, out0_ref, bns_ref, bnq_ref, *, d):
    i = pl.program_id(0)
    deg = deg_ref[...]
    degc = jnp.maximum(deg, 1.0)
    b = ab_ref[:, d:2 * d]
    s1 = s1_ref[...]
    s2 = s2_ref[...]
    pos = deg > 0.0
    summ = s1 + deg * b
    mean = summ / degc
    mx = jnp.where(pos, mx_ref[...] + b, 0.0)
    mn = jnp.where(pos, mn_ref[...] + b, 0.0)
    mean_sq = (s2 + 2.0 * b * s1 + deg * b * b) / degc
    std = jnp.sqrt(jax.nn.relu(mean_sq - mean * mean) + _EPS)
    agg = jnp.concatenate([mean, mx, mn, std], axis=1)
    sc = jnp.log(degc + 1.0) / _AVG_D_LOG
    hc = jnp.concatenate([h_ref[...], agg, agg * sc, agg / sc], axis=1)
    out0 = jax.lax.dot_general(
        hc, wp_ref[...], (((1,), (0,)), ((), ())),
        preferred_element_type=jnp.float32) + bp_ref[...]
    out0 = out0 * sn_ref[...]
    out0_ref[...] = out0

    @pl.when(i == 0)
    def _():
        bns_ref[...] = jnp.zeros_like(bns_ref)
        bnq_ref[...] = jnp.zeros_like(bnq_ref)

    bns_ref[...] += jnp.sum(out0, axis=0, keepdims=True)
    bnq_ref[...] += jnp.sum(out0 * out0, axis=0, keepdims=True)


def _bn_body(out0_ref, bns_ref, bnq_ref, g_ref, be_ref, out_ref, *, n):
    mu = bns_ref[...] / n
    var = bnq_ref[...] / n - mu * mu
    inv = jax.lax.rsqrt(var + _EPS)
    out_ref[...] = (out0_ref[...] - mu) * inv * g_ref[...] + be_ref[...]


def kernel(h, edge_index, e, snorm_n, W_pre, b_pre, W_post, b_post, gamma,
           beta):
    n, d = h.shape
    e_cnt = edge_index.shape[1]
    ed = e.shape[1]

    w12 = W_pre[:2 * d]
    w1 = W_pre[:d]
    w2 = W_pre[d:2 * d]
    w3 = W_pre[2 * d:]
    wab = jnp.concatenate([w1, w2], axis=1)
    b_pre2 = b_pre.reshape(1, d)
    b_post2 = b_post.reshape(1, d)
    gamma2 = gamma.reshape(1, d)
    beta2 = beta.reshape(1, d)

    nb_n = 10
    blk_n = n // nb_n

    ab = pl.pallas_call(
        _ab_body,
        grid=(nb_n,),
        in_specs=[pl.BlockSpec((blk_n, d), lambda i: (i, 0)),
                  pl.BlockSpec((d, 2 * d), lambda i: (0, 0))],
        out_specs=pl.BlockSpec((blk_n, 2 * d), lambda i: (i, 0)),
        out_shape=jax.ShapeDtypeStruct((n, 2 * d), jnp.float32),
    )(h, wab)

    eb_c = 4000
    nb_c = e_cnt // eb_c
    c = pl.pallas_call(
        _c_body,
        grid=(nb_c,),
        in_specs=[pl.BlockSpec((eb_c, ed), lambda i: (i, 0)),
                  pl.BlockSpec((ed, d), lambda i: (0, 0)),
                  pl.BlockSpec((1, d), lambda i: (0, 0))],
        out_specs=pl.BlockSpec((eb_c, d), lambda i: (i, 0)),
        out_shape=jax.ShapeDtypeStruct((e_cnt, d), jnp.float32),
    )(e, w3, b_pre2)

    eb = 1280
    nb_e = e_cnt // eb
    ei3 = edge_index.reshape(2, nb_e, eb).transpose(1, 0, 2)
    a = ab[:, :d]

    s1, s2, mx, mn, deg = pl.pallas_call(
        functools.partial(_seg_body, eb=eb),
        grid=(nb_e,),
        in_specs=[
            pl.BlockSpec((1, 2, eb), lambda i: (i, 0, 0),
                         memory_space=pltpu.SMEM),
            pl.BlockSpec((n, d), lambda i: (0, 0)),
            pl.BlockSpec((eb, d), lambda i: (i, 0)),
        ],
        out_specs=[
            pl.BlockSpec((n, d), lambda i: (0, 0)),
            pl.BlockSpec((n, d), lambda i: (0, 0)),
            pl.BlockSpec((n, d), lambda i: (0, 0)),
            pl.BlockSpec((n, d), lambda i: (0, 0)),
            pl.BlockSpec((n, 1), lambda i: (0, 0)),
        ],
        out_shape=[
            jax.ShapeDtypeStruct((n, d), jnp.float32),
            jax.ShapeDtypeStruct((n, d), jnp.float32),
            jax.ShapeDtypeStruct((n, d), jnp.float32),
            jax.ShapeDtypeStruct((n, d), jnp.float32),
            jax.ShapeDtypeStruct((n, 1), jnp.float32),
        ],
    )(ei3, a, c)

    out0, bns, bnq = pl.pallas_call(
        functools.partial(_post_body, d=d),
        grid=(nb_n,),
        in_specs=[
            pl.BlockSpec((blk_n, d), lambda i: (i, 0)),
            pl.BlockSpec((blk_n, 2 * d), lambda i: (i, 0)),
            pl.BlockSpec((blk_n, d), lambda i: (i, 0)),
            pl.BlockSpec((blk_n, d), lambda i: (i, 0)),
            pl.BlockSpec((blk_n, d), lambda i: (i, 0)),
            pl.BlockSpec((blk_n, d), lambda i: (i, 0)),
            pl.BlockSpec((blk_n, 1), lambda i: (i, 0)),
            pl.BlockSpec((blk_n, 1), lambda i: (i, 0)),
            pl.BlockSpec((13 * d, d), lambda i: (0, 0)),
            pl.BlockSpec((1, d), lambda i: (0, 0)),
        ],
        out_specs=[
            pl.BlockSpec((blk_n, d), lambda i: (i, 0)),
            pl.BlockSpec((1, d), lambda i: (0, 0)),
            pl.BlockSpec((1, d), lambda i: (0, 0)),
        ],
        out_shape=[
            jax.ShapeDtypeStruct((n, d), jnp.float32),
            jax.ShapeDtypeStruct((1, d), jnp.float32),
            jax.ShapeDtypeStruct((1, d), jnp.float32),
        ],
    )(h, ab, s1, s2, mx, mn, deg, snorm_n, W_post, b_post2)

    out = pl.pallas_call(
        functools.partial(_bn_body, n=float(n)),
        grid=(nb_n,),
        in_specs=[
            pl.BlockSpec((blk_n, d), lambda i: (i, 0)),
            pl.BlockSpec((1, d), lambda i: (0, 0)),
            pl.BlockSpec((1, d), lambda i: (0, 0)),
            pl.BlockSpec((1, d), lambda i: (0, 0)),
            pl.BlockSpec((1, d), lambda i: (0, 0)),
        ],
        out_specs=pl.BlockSpec((blk_n, d), lambda i: (i, 0)),
        out_shape=jax.ShapeDtypeStruct((n, d), jnp.float32),
    )(out0, bns, bnq, gamma2, beta2)
    return out


# speedup vs baseline: 1.7082x; 1.1354x over previous
import dataclasses
import functools

import jax
import jax.numpy as jnp
from jax import lax
from jax.experimental import pallas as pl
from jax.experimental.pallas import tpu as pltpu
from jax.experimental.pallas import tpu_sc as plsc

_EPS = 1e-5
_AVG_D_LOG = 3.5

_NW = 32
_NPW = 320
_NPC = 160
_NPAD = _NW * _NPW
_T = 2560
_GB = 64


def _ab_body(h_ref, w_ref, a_ref, b_ref):
    ab = jax.lax.dot_general(
        h_ref[...], w_ref[...], (((1,), (0,)), ((), ())),
        preferred_element_type=jnp.float32)
    d = a_ref.shape[1]
    a_ref[...] = ab[:, :d]
    b_ref[...] = ab[:, d:]


def _c_body(e_ref, w_ref, b_ref, c_ref):
    c_ref[...] = jax.lax.dot_general(
        e_ref[...], w_ref[...], (((1,), (0,)), ((), ())),
        preferred_element_type=jnp.float32) + b_ref[...]


def _sc_seg_body(src_hbm, dst_hbm, a_hbm, c_hbm,
                 s1_hbm, s2_hbm, mx_hbm, mn_hbm, deg_hbm,
                 s1_v, s2_v, mx_v, mn_v, deg_v, dstt_v, eid_v, rel_v,
                 srcv_v, arow_v, crow_v, *, n_tiles, d):
    c = lax.axis_index("c")
    s = lax.axis_index("s")
    wid = c * 16 + s
    nch = d // 16
    iota = lax.iota(jnp.int32, 16)
    zeros16 = jnp.zeros((16,), jnp.float32)
    ninf16 = jnp.full((16,), -jnp.inf, jnp.float32)
    pinf16 = jnp.full((16,), jnp.inf, jnp.float32)
    izeros16 = jnp.zeros((16,), jnp.int32)

    @pl.loop(0, (_T + 16) // 16)
    def _(r):
        eid_v[pl.ds(r * 16, 16)] = izeros16
        rel_v[pl.ds(r * 16, 16)] = izeros16

    for g in range(2):
        lo = wid * _NPW + g * _NPC

        @pl.loop(0, _NPC)
        def _(r):
            for ch in range(nch):
                sl = pl.ds(ch * 16, 16)
                s1_v[r, sl] = zeros16
                s2_v[r, sl] = zeros16
                mx_v[r, sl] = ninf16
                mn_v[r, sl] = pinf16
            deg_v[r, :] = zeros16

        def tile_body(t, carry):
            tb = t * _T
            pltpu.sync_copy(dst_hbm.at[pl.ds(tb, _T)], dstt_v)

            def scan_body(j, cnt):
                for gg in range(4):
                    dv = dstt_v[pl.ds((j * 4 + gg) * 16, 16)]
                    rel = dv - lo
                    m = (rel >= 0) & (rel < _NPC)
                    eidv = iota + (tb + (j * 4 + gg) * 16)
                    plsc.store_compressed(eid_v.at[pl.ds(cnt, 16)], eidv,
                                          mask=m)
                    plsc.store_compressed(rel_v.at[pl.ds(cnt, 16)], rel,
                                          mask=m)
                    cnt = cnt + jnp.sum(m.astype(jnp.int32))
                return cnt

            cnt = lax.fori_loop(0, _T // 64, scan_body, 0)

            def batch_body(b, carry):
                k = b * _GB
                kb = jnp.minimum(_GB, cnt - k)
                pltpu.sync_copy(src_hbm.at[eid_v.at[pl.ds(k, _GB)]], srcv_v)
                pltpu.sync_copy(a_hbm.at[srcv_v], arow_v)
                pltpu.sync_copy(c_hbm.at[eid_v.at[pl.ds(k, _GB)]], crow_v)

                def edge_body(i, carry):
                    ri = rel_v[pl.ds(k + i, 16)][0]
                    for ch in range(nch):
                        sl = pl.ds(ch * 16, 16)
                        u = arow_v[i, sl] + crow_v[i, sl]
                        plsc.addupdate(s1_v.at[ri, sl], u)
                        plsc.addupdate(s2_v.at[ri, sl], u * u)
                        mx_v[ri, sl] = jnp.maximum(mx_v[ri, sl], u)
                        mn_v[ri, sl] = jnp.minimum(mn_v[ri, sl], u)
                    deg_v[ri, :] = deg_v[ri, :] + 1.0
                    return carry

                lax.fori_loop(0, kb, edge_body, 0)
                return carry

            lax.fori_loop(0, (cnt + _GB - 1) // _GB, batch_body, 0)
            return carry

        lax.fori_loop(0, n_tiles, tile_body, 0)

        pltpu.sync_copy(s1_v, s1_hbm.at[pl.ds(lo, _NPC)])
        pltpu.sync_copy(s2_v, s2_hbm.at[pl.ds(lo, _NPC)])
        pltpu.sync_copy(mx_v, mx_hbm.at[pl.ds(lo, _NPC)])
        pltpu.sync_copy(mn_v, mn_hbm.at[pl.ds(lo, _NPC)])
        pltpu.sync_copy(deg_v, deg_hbm.at[pl.ds(lo, _NPC)])


def _post_body(h_ref, b_ref, s1_ref, s2_ref, mx_ref, mn_ref, deg_ref,
               sn_ref, wp_ref, bp_ref, out0_ref, bns_ref, bnq_ref, *, d):
    i = pl.program_id(0)
    deg = deg_ref[:, :1]
    degc = jnp.maximum(deg, 1.0)
    b = b_ref[...]
    s1 = s1_ref[...]
    s2 = s2_ref[...]
    pos = deg > 0.0
    summ = s1 + deg * b
    mean = summ / degc
    mx = jnp.where(pos, mx_ref[...] + b, 0.0)
    mn = jnp.where(pos, mn_ref[...] + b, 0.0)
    mean_sq = (s2 + 2.0 * b * s1 + deg * b * b) / degc
    std = jnp.sqrt(jax.nn.relu(mean_sq - mean * mean) + _EPS)
    agg = jnp.concatenate([mean, mx, mn, std], axis=1)
    sc = jnp.log(degc + 1.0) / _AVG_D_LOG
    hc = jnp.concatenate([h_ref[...], agg, agg * sc, agg / sc], axis=1)
    out0 = jax.lax.dot_general(
        hc, wp_ref[...], (((1,), (0,)), ((), ())),
        preferred_element_type=jnp.float32) + bp_ref[...]
    out0 = out0 * sn_ref[...]
    out0_ref[...] = out0

    @pl.when(i == 0)
    def _():
        bns_ref[...] = jnp.zeros_like(bns_ref)
        bnq_ref[...] = jnp.zeros_like(bnq_ref)

    bns_ref[...] += jnp.sum(out0, axis=0, keepdims=True)
    bnq_ref[...] += jnp.sum(out0 * out0, axis=0, keepdims=True)


def _bn_body(out0_ref, bns_ref, bnq_ref, g_ref, be_ref, out_ref, *, n):
    mu = bns_ref[...] / n
    var = bnq_ref[...] / n - mu * mu
    inv = jax.lax.rsqrt(var + _EPS)
    out_ref[...] = (out0_ref[...] - mu) * inv * g_ref[...] + be_ref[...]


def kernel(h, edge_index, e, snorm_n, W_pre, b_pre, W_post, b_post, gamma,
           beta):
    n, d = h.shape
    e_cnt = edge_index.shape[1]
    ed = e.shape[1]

    w12 = W_pre[:2 * d]
    w1 = W_pre[:d]
    w2 = W_pre[d:2 * d]
    w3 = W_pre[2 * d:]
    wab = jnp.concatenate([w1, w2], axis=1)
    b_pre2 = b_pre.reshape(1, d)
    b_post2 = b_post.reshape(1, d)
    gamma2 = gamma.reshape(1, d)
    beta2 = beta.reshape(1, d)

    nb_n = 10
    blk_n = n // nb_n

    a, b = pl.pallas_call(
        _ab_body,
        grid=(nb_n,),
        in_specs=[pl.BlockSpec((blk_n, d), lambda i: (i, 0)),
                  pl.BlockSpec((d, 2 * d), lambda i: (0, 0))],
        out_specs=[pl.BlockSpec((blk_n, d), lambda i: (i, 0)),
                   pl.BlockSpec((blk_n, d), lambda i: (i, 0))],
        out_shape=[jax.ShapeDtypeStruct((n, d), jnp.float32),
                   jax.ShapeDtypeStruct((n, d), jnp.float32)],
    )(h, wab)

    eb_c = 4000
    nb_c = e_cnt // eb_c
    c = pl.pallas_call(
        _c_body,
        grid=(nb_c,),
        in_specs=[pl.BlockSpec((eb_c, ed), lambda i: (i, 0)),
                  pl.BlockSpec((ed, d), lambda i: (0, 0)),
                  pl.BlockSpec((1, d), lambda i: (0, 0))],
        out_specs=pl.BlockSpec((eb_c, d), lambda i: (i, 0)),
        out_shape=jax.ShapeDtypeStruct((e_cnt, d), jnp.float32),
    )(e, w3, b_pre2)

    src = edge_index[0]
    dst = edge_index[1]
    n_tiles = e_cnt // _T

    f32 = jnp.float32
    sc_params = pltpu.CompilerParams()
    if "needs_layout_passes" in pltpu.CompilerParams.__dataclass_fields__:
        sc_params = dataclasses.replace(sc_params, needs_layout_passes=False)
    seg_kernel = pl.kernel(
        functools.partial(_sc_seg_body, n_tiles=n_tiles, d=d),
        mesh=plsc.VectorSubcoreMesh(core_axis_name="c", subcore_axis_name="s"),
        out_type=[
            jax.ShapeDtypeStruct((_NPAD, d), f32),
            jax.ShapeDtypeStruct((_NPAD, d), f32),
            jax.ShapeDtypeStruct((_NPAD, d), f32),
            jax.ShapeDtypeStruct((_NPAD, d), f32),
            jax.ShapeDtypeStruct((_NPAD, 16), f32),
        ],
        scratch_types=[
            pltpu.VMEM((_NPC, d), f32),
            pltpu.VMEM((_NPC, d), f32),
            pltpu.VMEM((_NPC, d), f32),
            pltpu.VMEM((_NPC, d), f32),
            pltpu.VMEM((_NPC, 16), f32),
            pltpu.VMEM((_T,), jnp.int32),
            pltpu.VMEM((_T + 16,), jnp.int32),
            pltpu.VMEM((_T + 16,), jnp.int32),
            pltpu.VMEM((_GB,), jnp.int32),
            pltpu.VMEM((_GB, d), f32),
            pltpu.VMEM((_GB, d), f32),
        ],
        compiler_params=sc_params,
    )
    s1, s2, mx, mn, deg = seg_kernel(src, dst, a, c)
    s1, s2, mx, mn, deg = (s1[:n], s2[:n], mx[:n], mn[:n], deg[:n])

    out0, bns, bnq = pl.pallas_call(
        functools.partial(_post_body, d=d),
        grid=(nb_n,),
        in_specs=[
            pl.BlockSpec((blk_n, d), lambda i: (i, 0)),
            pl.BlockSpec((blk_n, d), lambda i: (i, 0)),
            pl.BlockSpec((blk_n, d), lambda i: (i, 0)),
            pl.BlockSpec((blk_n, d), lambda i: (i, 0)),
            pl.BlockSpec((blk_n, d), lambda i: (i, 0)),
            pl.BlockSpec((blk_n, d), lambda i: (i, 0)),
            pl.BlockSpec((blk_n, 16), lambda i: (i, 0)),
            pl.BlockSpec((blk_n, 1), lambda i: (i, 0)),
            pl.BlockSpec((13 * d, d), lambda i: (0, 0)),
            pl.BlockSpec((1, d), lambda i: (0, 0)),
        ],
        out_specs=[
            pl.BlockSpec((blk_n, d), lambda i: (i, 0)),
            pl.BlockSpec((1, d), lambda i: (0, 0)),
            pl.BlockSpec((1, d), lambda i: (0, 0)),
        ],
        out_shape=[
            jax.ShapeDtypeStruct((n, d), jnp.float32),
            jax.ShapeDtypeStruct((1, d), jnp.float32),
            jax.ShapeDtypeStruct((1, d), jnp.float32),
        ],
    )(h, b, s1, s2, mx, mn, deg, snorm_n, W_post, b_post2)

    out = pl.pallas_call(
        functools.partial(_bn_body, n=float(n)),
        grid=(nb_n,),
        in_specs=[
            pl.BlockSpec((blk_n, d), lambda i: (i, 0)),
            pl.BlockSpec((1, d), lambda i: (0, 0)),
            pl.BlockSpec((1, d), lambda i: (0, 0)),
            pl.BlockSpec((1, d), lambda i: (0, 0)),
            pl.BlockSpec((1, d), lambda i: (0, 0)),
        ],
        out_specs=pl.BlockSpec((blk_n, d), lambda i: (i, 0)),
        out_shape=jax.ShapeDtypeStruct((n, d), jnp.float32),
    )(out0, bns, bnq, gamma2, beta2)
    return out


# speedup vs baseline: 1.7877x; 1.0465x over previous
import dataclasses
import functools

import jax
import jax.numpy as jnp
from jax import lax
from jax.experimental import pallas as pl
from jax.experimental.pallas import tpu as pltpu
from jax.experimental.pallas import tpu_sc as plsc

_EPS = 1e-5
_AVG_D_LOG = 3.5

_NW = 32
_NPW = 320
_NPC = 160
_NPAD = _NW * _NPW
_T = 3200
_GB = 56


def _ab_body(h_ref, w_ref, a_ref, b_ref):
    ab = jax.lax.dot_general(
        h_ref[...], w_ref[...], (((1,), (0,)), ((), ())),
        preferred_element_type=jnp.float32)
    d = a_ref.shape[1]
    a_ref[...] = ab[:, :d]
    b_ref[...] = ab[:, d:]


def _c_body(e_ref, w_ref, b_ref, c_ref):
    c_ref[...] = jax.lax.dot_general(
        e_ref[...], w_ref[...], (((1,), (0,)), ((), ())),
        preferred_element_type=jnp.float32) + b_ref[...]


def _sc_seg_body(src_hbm, dst_hbm, a_hbm, c_hbm,
                 s1_hbm, s2_hbm, mx_hbm, mn_hbm, deg_hbm,
                 s1_v, s2_v, mx_v, mn_v, deg_v, dstt0_v, dstt1_v,
                 eid_v, rel_v, srcv_v, arow_v, crow_v,
                 sem0, sem1, semg1, semg2, *, n_tiles, d):
    c = lax.axis_index("c")
    s = lax.axis_index("s")
    wid = c * 16 + s
    nch = d // 16
    iota = lax.iota(jnp.int32, 16)
    zeros16 = jnp.zeros((16,), jnp.float32)
    ninf16 = jnp.full((16,), -jnp.inf, jnp.float32)
    pinf16 = jnp.full((16,), jnp.inf, jnp.float32)
    izeros16 = jnp.zeros((16,), jnp.int32)

    @pl.loop(0, (_T + 16) // 16)
    def _(r):
        eid_v[pl.ds(r * 16, 16)] = izeros16
        rel_v[pl.ds(r * 16, 16)] = izeros16

    for g in range(2):
        lo = wid * _NPW + g * _NPC

        @pl.loop(0, _NPC)
        def _(r):
            for ch in range(nch):
                sl = pl.ds(ch * 16, 16)
                s1_v[r, sl] = zeros16
                s2_v[r, sl] = zeros16
                mx_v[r, sl] = ninf16
                mn_v[r, sl] = pinf16
            deg_v[r, :] = zeros16

        def process_tile(t, dstt_v):
            tb = t * _T

            def scan_body(j, cnt):
                for gg in range(4):
                    dv = dstt_v[pl.ds((j * 4 + gg) * 16, 16)]
                    rel = dv - lo
                    m = (rel >= 0) & (rel < _NPC)
                    eidv = iota + (tb + (j * 4 + gg) * 16)
                    plsc.store_compressed(eid_v.at[pl.ds(cnt, 16)], eidv,
                                          mask=m)
                    plsc.store_compressed(rel_v.at[pl.ds(cnt, 16)], rel,
                                          mask=m)
                    cnt = cnt + jnp.sum(m.astype(jnp.int32))
                return cnt

            cnt = lax.fori_loop(0, _T // 64, scan_body, 0)

            def batch_body(b, carry):
                k = b * _GB
                kb = jnp.minimum(_GB, cnt - k)
                cp_s = pltpu.async_copy(
                    src_hbm.at[eid_v.at[pl.ds(k, _GB)]], srcv_v, semg1)
                cp_c = pltpu.async_copy(
                    c_hbm.at[eid_v.at[pl.ds(k, _GB)]], crow_v, semg2)
                cp_s.wait()
                cp_a = pltpu.async_copy(a_hbm.at[srcv_v], arow_v, semg1)
                cp_a.wait()
                cp_c.wait()

                def edge_body(i, carry):
                    ri = rel_v[pl.ds(k + i, 16)][0]
                    for ch in range(nch):
                        sl = pl.ds(ch * 16, 16)
                        u = arow_v[i, sl] + crow_v[i, sl]
                        plsc.addupdate(s1_v.at[ri, sl], u)
                        plsc.addupdate(s2_v.at[ri, sl], u * u)
                        mx_v[ri, sl] = jnp.maximum(mx_v[ri, sl], u)
                        mn_v[ri, sl] = jnp.minimum(mn_v[ri, sl], u)
                    deg_v[ri, :] = deg_v[ri, :] + 1.0
                    return carry

                lax.fori_loop(0, kb, edge_body, 0)
                return carry

            lax.fori_loop(0, (cnt + _GB - 1) // _GB, batch_body, 0)

        def start_fetch(t, dstt_v, sem):
            pltpu.async_copy(dst_hbm.at[pl.ds(t * _T, _T)], dstt_v, sem)

        def wait_fetch(dstt_v, sem):
            pltpu.make_async_copy(dst_hbm.at[pl.ds(0, _T)], dstt_v, sem).wait()

        start_fetch(0, dstt0_v, sem0)

        def pair_body(j, carry):
            t0 = j * 2
            wait_fetch(dstt0_v, sem0)
            start_fetch(t0 + 1, dstt1_v, sem1)
            process_tile(t0, dstt0_v)

            @pl.when(t0 + 2 < n_tiles)
            def _():
                start_fetch(t0 + 2, dstt0_v, sem0)

            wait_fetch(dstt1_v, sem1)
            process_tile(t0 + 1, dstt1_v)
            return carry

        lax.fori_loop(0, n_tiles // 2, pair_body, 0)

        pltpu.sync_copy(s1_v, s1_hbm.at[pl.ds(lo, _NPC)])
        pltpu.sync_copy(s2_v, s2_hbm.at[pl.ds(lo, _NPC)])
        pltpu.sync_copy(mx_v, mx_hbm.at[pl.ds(lo, _NPC)])
        pltpu.sync_copy(mn_v, mn_hbm.at[pl.ds(lo, _NPC)])
        pltpu.sync_copy(deg_v, deg_hbm.at[pl.ds(lo, _NPC)])


def _post_body(h_ref, b_ref, s1_ref, s2_ref, mx_ref, mn_ref, deg_ref,
               sn_ref, wp_ref, bp_ref, out0_ref, bns_ref, bnq_ref, *, d):
    i = pl.program_id(0)
    deg = deg_ref[:, :1]
    degc = jnp.maximum(deg, 1.0)
    b = b_ref[...]
    s1 = s1_ref[...]
    s2 = s2_ref[...]
    pos = deg > 0.0
    summ = s1 + deg * b
    mean = summ / degc
    mx = jnp.where(pos, mx_ref[...] + b, 0.0)
    mn = jnp.where(pos, mn_ref[...] + b, 0.0)
    mean_sq = (s2 + 2.0 * b * s1 + deg * b * b) / degc
    std = jnp.sqrt(jax.nn.relu(mean_sq - mean * mean) + _EPS)
    agg = jnp.concatenate([mean, mx, mn, std], axis=1)
    sc = jnp.log(degc + 1.0) / _AVG_D_LOG
    hc = jnp.concatenate([h_ref[...], agg, agg * sc, agg / sc], axis=1)
    out0 = jax.lax.dot_general(
        hc, wp_ref[...], (((1,), (0,)), ((), ())),
        preferred_element_type=jnp.float32) + bp_ref[...]
    out0 = out0 * sn_ref[...]
    out0_ref[...] = out0

    @pl.when(i == 0)
    def _():
        bns_ref[...] = jnp.zeros_like(bns_ref)
        bnq_ref[...] = jnp.zeros_like(bnq_ref)

    bns_ref[...] += jnp.sum(out0, axis=0, keepdims=True)
    bnq_ref[...] += jnp.sum(out0 * out0, axis=0, keepdims=True)


def _bn_body(out0_ref, bns_ref, bnq_ref, g_ref, be_ref, out_ref, *, n):
    mu = bns_ref[...] / n
    var = bnq_ref[...] / n - mu * mu
    inv = jax.lax.rsqrt(var + _EPS)
    out_ref[...] = (out0_ref[...] - mu) * inv * g_ref[...] + be_ref[...]


def kernel(h, edge_index, e, snorm_n, W_pre, b_pre, W_post, b_post, gamma,
           beta):
    n, d = h.shape
    e_cnt = edge_index.shape[1]
    ed = e.shape[1]

    w12 = W_pre[:2 * d]
    w1 = W_pre[:d]
    w2 = W_pre[d:2 * d]
    w3 = W_pre[2 * d:]
    wab = jnp.concatenate([w1, w2], axis=1)
    b_pre2 = b_pre.reshape(1, d)
    b_post2 = b_post.reshape(1, d)
    gamma2 = gamma.reshape(1, d)
    beta2 = beta.reshape(1, d)

    nb_n = 10
    blk_n = n // nb_n

    a, b = pl.pallas_call(
        _ab_body,
        grid=(nb_n,),
        in_specs=[pl.BlockSpec((blk_n, d), lambda i: (i, 0)),
                  pl.BlockSpec((d, 2 * d), lambda i: (0, 0))],
        out_specs=[pl.BlockSpec((blk_n, d), lambda i: (i, 0)),
                   pl.BlockSpec((blk_n, d), lambda i: (i, 0))],
        out_shape=[jax.ShapeDtypeStruct((n, d), jnp.float32),
                   jax.ShapeDtypeStruct((n, d), jnp.float32)],
    )(h, wab)

    eb_c = 4000
    nb_c = e_cnt // eb_c
    c = pl.pallas_call(
        _c_body,
        grid=(nb_c,),
        in_specs=[pl.BlockSpec((eb_c, ed), lambda i: (i, 0)),
                  pl.BlockSpec((ed, d), lambda i: (0, 0)),
                  pl.BlockSpec((1, d), lambda i: (0, 0))],
        out_specs=pl.BlockSpec((eb_c, d), lambda i: (i, 0)),
        out_shape=jax.ShapeDtypeStruct((e_cnt, d), jnp.float32),
    )(e, w3, b_pre2)

    src = edge_index[0]
    dst = edge_index[1]
    n_tiles = e_cnt // _T

    f32 = jnp.float32
    sc_params = pltpu.CompilerParams()
    if "needs_layout_passes" in pltpu.CompilerParams.__dataclass_fields__:
        sc_params = dataclasses.replace(sc_params, needs_layout_passes=False)
    seg_kernel = pl.kernel(
        functools.partial(_sc_seg_body, n_tiles=n_tiles, d=d),
        mesh=plsc.VectorSubcoreMesh(core_axis_name="c", subcore_axis_name="s"),
        out_type=[
            jax.ShapeDtypeStruct((_NPAD, d), f32),
            jax.ShapeDtypeStruct((_NPAD, d), f32),
            jax.ShapeDtypeStruct((_NPAD, d), f32),
            jax.ShapeDtypeStruct((_NPAD, d), f32),
            jax.ShapeDtypeStruct((_NPAD, 16), f32),
        ],
        scratch_types=[
            pltpu.VMEM((_NPC, d), f32),
            pltpu.VMEM((_NPC, d), f32),
            pltpu.VMEM((_NPC, d), f32),
            pltpu.VMEM((_NPC, d), f32),
            pltpu.VMEM((_NPC, 16), f32),
            pltpu.VMEM((_T,), jnp.int32),
            pltpu.VMEM((_T,), jnp.int32),
            pltpu.VMEM((_T + 16,), jnp.int32),
            pltpu.VMEM((_T + 16,), jnp.int32),
            pltpu.VMEM((_GB,), jnp.int32),
            pltpu.VMEM((_GB, d), f32),
            pltpu.VMEM((_GB, d), f32),
            pltpu.SemaphoreType.DMA,
            pltpu.SemaphoreType.DMA,
            pltpu.SemaphoreType.DMA,
            pltpu.SemaphoreType.DMA,
        ],
        compiler_params=sc_params,
    )
    s1, s2, mx, mn, deg = seg_kernel(src, dst, a, c)
    s1, s2, mx, mn, deg = (s1[:n], s2[:n], mx[:n], mn[:n], deg[:n])

    out0, bns, bnq = pl.pallas_call(
        functools.partial(_post_body, d=d),
        grid=(nb_n,),
        in_specs=[
            pl.BlockSpec((blk_n, d), lambda i: (i, 0)),
            pl.BlockSpec((blk_n, d), lambda i: (i, 0)),
            pl.BlockSpec((blk_n, d), lambda i: (i, 0)),
            pl.BlockSpec((blk_n, d), lambda i: (i, 0)),
            pl.BlockSpec((blk_n, d), lambda i: (i, 0)),
            pl.BlockSpec((blk_n, d), lambda i: (i, 0)),
            pl.BlockSpec((blk_n, 16), lambda i: (i, 0)),
            pl.BlockSpec((blk_n, 1), lambda i: (i, 0)),
            pl.BlockSpec((13 * d, d), lambda i: (0, 0)),
            pl.BlockSpec((1, d), lambda i: (0, 0)),
        ],
        out_specs=[
            pl.BlockSpec((blk_n, d), lambda i: (i, 0)),
            pl.BlockSpec((1, d), lambda i: (0, 0)),
            pl.BlockSpec((1, d), lambda i: (0, 0)),
        ],
        out_shape=[
            jax.ShapeDtypeStruct((n, d), jnp.float32),
            jax.ShapeDtypeStruct((1, d), jnp.float32),
            jax.ShapeDtypeStruct((1, d), jnp.float32),
        ],
    )(h, b, s1, s2, mx, mn, deg, snorm_n, W_post, b_post2)

    out = pl.pallas_call(
        functools.partial(_bn_body, n=float(n)),
        grid=(nb_n,),
        in_specs=[
            pl.BlockSpec((blk_n, d), lambda i: (i, 0)),
            pl.BlockSpec((1, d), lambda i: (0, 0)),
            pl.BlockSpec((1, d), lambda i: (0, 0)),
            pl.BlockSpec((1, d), lambda i: (0, 0)),
            pl.BlockSpec((1, d), lambda i: (0, 0)),
        ],
        out_specs=pl.BlockSpec((blk_n, d), lambda i: (i, 0)),
        out_shape=jax.ShapeDtypeStruct((n, d), jnp.float32),
    )(out0, bns, bnq, gamma2, beta2)
    return out


# speedup vs baseline: 1.7891x; 1.0008x over previous
import dataclasses
import functools

import jax
import jax.numpy as jnp
from jax import lax
from jax.experimental import pallas as pl
from jax.experimental.pallas import tpu as pltpu
from jax.experimental.pallas import tpu_sc as plsc

_EPS = 1e-5
_AVG_D_LOG = 3.5

_NW = 32
_NPW = 320
_NPC = 160
_NPAD = _NW * _NPW
_T = 3200
_GB = 56


def _ab_body(h_ref, w_ref, a_ref, b_ref):
    ab = jax.lax.dot_general(
        h_ref[...], w_ref[...], (((1,), (0,)), ((), ())),
        preferred_element_type=jnp.float32)
    d = a_ref.shape[1]
    a_ref[...] = ab[:, :d]
    b_ref[...] = ab[:, d:]


def _c_body(e_ref, w_ref, b_ref, c_ref):
    c_ref[...] = jax.lax.dot_general(
        e_ref[...], w_ref[...], (((1,), (0,)), ((), ())),
        preferred_element_type=jnp.float32) + b_ref[...]


def _sc_seg_body(src_hbm, dst_hbm, a_hbm, c_hbm,
                 s1_hbm, s2_hbm, mx_hbm, mn_hbm, deg_hbm,
                 s1_v, s2_v, mx_v, mn_v, deg_v, dstt0_v, dstt1_v,
                 eid_v, rel_v, srcv_v, arow_v, crow_v,
                 sem0, sem1, semg1, semg2, *, n_tiles, d):
    c = lax.axis_index("c")
    s = lax.axis_index("s")
    wid = c * 16 + s
    nch = d // 16
    iota = lax.iota(jnp.int32, 16)
    zeros16 = jnp.zeros((16,), jnp.float32)
    ninf16 = jnp.full((16,), -jnp.inf, jnp.float32)
    pinf16 = jnp.full((16,), jnp.inf, jnp.float32)
    izeros16 = jnp.zeros((16,), jnp.int32)

    @pl.loop(0, (_T + 16) // 16)
    def _(r):
        eid_v[pl.ds(r * 16, 16)] = izeros16
        rel_v[pl.ds(r * 16, 16)] = izeros16

    for g in range(2):
        lo = wid * _NPW + g * _NPC

        @pl.loop(0, _NPC)
        def _(r):
            for ch in range(nch):
                sl = pl.ds(ch * 16, 16)
                s1_v[r, sl] = zeros16
                s2_v[r, sl] = zeros16
                mx_v[r, sl] = ninf16
                mn_v[r, sl] = pinf16
            deg_v[r, :] = zeros16

        def process_tile(t, dstt_v):
            tb = t * _T

            def scan_body(j, cnt):
                for gg in range(4):
                    dv = dstt_v[pl.ds((j * 4 + gg) * 16, 16)]
                    rel = dv - lo
                    m = (rel >= 0) & (rel < _NPC)
                    eidv = iota + (tb + (j * 4 + gg) * 16)
                    plsc.store_compressed(eid_v.at[pl.ds(cnt, 16)], eidv,
                                          mask=m)
                    plsc.store_compressed(rel_v.at[pl.ds(cnt, 16)], rel,
                                          mask=m)
                    cnt = cnt + plsc.all_reduce_population_count(m)[0]
                return cnt

            cnt = lax.fori_loop(0, _T // 64, scan_body, 0)

            def batch_body(b, carry):
                k = b * _GB
                kb = jnp.minimum(_GB, cnt - k)
                cp_s = pltpu.async_copy(
                    src_hbm.at[eid_v.at[pl.ds(k, _GB)]], srcv_v, semg1)
                cp_c = pltpu.async_copy(
                    c_hbm.at[eid_v.at[pl.ds(k, _GB)]], crow_v, semg2)
                cp_s.wait()
                cp_a = pltpu.async_copy(a_hbm.at[srcv_v], arow_v, semg1)
                cp_a.wait()
                cp_c.wait()

                def edge_body(i, carry):
                    ri = rel_v[pl.ds(k + i, 16)][0]
                    for ch in range(nch):
                        sl = pl.ds(ch * 16, 16)
                        u = arow_v[i, sl] + crow_v[i, sl]
                        plsc.addupdate(s1_v.at[ri, sl], u)
                        plsc.addupdate(s2_v.at[ri, sl], u * u)
                        mx_v[ri, sl] = jnp.maximum(mx_v[ri, sl], u)
                        mn_v[ri, sl] = jnp.minimum(mn_v[ri, sl], u)
                    deg_v[ri, :] = deg_v[ri, :] + 1.0
                    return carry

                lax.fori_loop(0, kb, edge_body, 0)
                return carry

            lax.fori_loop(0, (cnt + _GB - 1) // _GB, batch_body, 0)

        def start_fetch(t, dstt_v, sem):
            pltpu.async_copy(dst_hbm.at[pl.ds(t * _T, _T)], dstt_v, sem)

        def wait_fetch(dstt_v, sem):
            pltpu.make_async_copy(dst_hbm.at[pl.ds(0, _T)], dstt_v, sem).wait()

        start_fetch(0, dstt0_v, sem0)

        def pair_body(j, carry):
            t0 = j * 2
            wait_fetch(dstt0_v, sem0)
            start_fetch(t0 + 1, dstt1_v, sem1)
            process_tile(t0, dstt0_v)

            @pl.when(t0 + 2 < n_tiles)
            def _():
                start_fetch(t0 + 2, dstt0_v, sem0)

            wait_fetch(dstt1_v, sem1)
            process_tile(t0 + 1, dstt1_v)
            return carry

        lax.fori_loop(0, n_tiles // 2, pair_body, 0)

        pltpu.sync_copy(s1_v, s1_hbm.at[pl.ds(lo, _NPC)])
        pltpu.sync_copy(s2_v, s2_hbm.at[pl.ds(lo, _NPC)])
        pltpu.sync_copy(mx_v, mx_hbm.at[pl.ds(lo, _NPC)])
        pltpu.sync_copy(mn_v, mn_hbm.at[pl.ds(lo, _NPC)])
        pltpu.sync_copy(deg_v, deg_hbm.at[pl.ds(lo, _NPC)])


def _post_body(h_ref, b_ref, s1_ref, s2_ref, mx_ref, mn_ref, deg_ref,
               sn_ref, wp_ref, bp_ref, out0_ref, bns_ref, bnq_ref, *, d):
    i = pl.program_id(0)
    deg = deg_ref[:, :1]
    degc = jnp.maximum(deg, 1.0)
    b = b_ref[...]
    s1 = s1_ref[...]
    s2 = s2_ref[...]
    pos = deg > 0.0
    summ = s1 + deg * b
    mean = summ / degc
    mx = jnp.where(pos, mx_ref[...] + b, 0.0)
    mn = jnp.where(pos, mn_ref[...] + b, 0.0)
    mean_sq = (s2 + 2.0 * b * s1 + deg * b * b) / degc
    std = jnp.sqrt(jax.nn.relu(mean_sq - mean * mean) + _EPS)
    agg = jnp.concatenate([mean, mx, mn, std], axis=1)
    sc = jnp.log(degc + 1.0) / _AVG_D_LOG
    hc = jnp.concatenate([h_ref[...], agg, agg * sc, agg / sc], axis=1)
    out0 = jax.lax.dot_general(
        hc, wp_ref[...], (((1,), (0,)), ((), ())),
        preferred_element_type=jnp.float32) + bp_ref[...]
    out0 = out0 * sn_ref[...]
    out0_ref[...] = out0

    @pl.when(i == 0)
    def _():
        bns_ref[...] = jnp.zeros_like(bns_ref)
        bnq_ref[...] = jnp.zeros_like(bnq_ref)

    bns_ref[...] += jnp.sum(out0, axis=0, keepdims=True)
    bnq_ref[...] += jnp.sum(out0 * out0, axis=0, keepdims=True)


def _bn_body(out0_ref, bns_ref, bnq_ref, g_ref, be_ref, out_ref, *, n):
    mu = bns_ref[...] / n
    var = bnq_ref[...] / n - mu * mu
    inv = jax.lax.rsqrt(var + _EPS)
    out_ref[...] = (out0_ref[...] - mu) * inv * g_ref[...] + be_ref[...]


def kernel(h, edge_index, e, snorm_n, W_pre, b_pre, W_post, b_post, gamma,
           beta):
    n, d = h.shape
    e_cnt = edge_index.shape[1]
    ed = e.shape[1]

    w12 = W_pre[:2 * d]
    w1 = W_pre[:d]
    w2 = W_pre[d:2 * d]
    w3 = W_pre[2 * d:]
    wab = jnp.concatenate([w1, w2], axis=1)
    b_pre2 = b_pre.reshape(1, d)
    b_post2 = b_post.reshape(1, d)
    gamma2 = gamma.reshape(1, d)
    beta2 = beta.reshape(1, d)

    nb_n = 10
    blk_n = n // nb_n

    a, b = pl.pallas_call(
        _ab_body,
        grid=(nb_n,),
        in_specs=[pl.BlockSpec((blk_n, d), lambda i: (i, 0)),
                  pl.BlockSpec((d, 2 * d), lambda i: (0, 0))],
        out_specs=[pl.BlockSpec((blk_n, d), lambda i: (i, 0)),
                   pl.BlockSpec((blk_n, d), lambda i: (i, 0))],
        out_shape=[jax.ShapeDtypeStruct((n, d), jnp.float32),
                   jax.ShapeDtypeStruct((n, d), jnp.float32)],
    )(h, wab)

    eb_c = 4000
    nb_c = e_cnt // eb_c
    c = pl.pallas_call(
        _c_body,
        grid=(nb_c,),
        in_specs=[pl.BlockSpec((eb_c, ed), lambda i: (i, 0)),
                  pl.BlockSpec((ed, d), lambda i: (0, 0)),
                  pl.BlockSpec((1, d), lambda i: (0, 0))],
        out_specs=pl.BlockSpec((eb_c, d), lambda i: (i, 0)),
        out_shape=jax.ShapeDtypeStruct((e_cnt, d), jnp.float32),
    )(e, w3, b_pre2)

    src = edge_index[0]
    dst = edge_index[1]
    n_tiles = e_cnt // _T

    f32 = jnp.float32
    sc_params = pltpu.CompilerParams()
    if "needs_layout_passes" in pltpu.CompilerParams.__dataclass_fields__:
        sc_params = dataclasses.replace(sc_params, needs_layout_passes=False)
    seg_kernel = pl.kernel(
        functools.partial(_sc_seg_body, n_tiles=n_tiles, d=d),
        mesh=plsc.VectorSubcoreMesh(core_axis_name="c", subcore_axis_name="s"),
        out_type=[
            jax.ShapeDtypeStruct((_NPAD, d), f32),
            jax.ShapeDtypeStruct((_NPAD, d), f32),
            jax.ShapeDtypeStruct((_NPAD, d), f32),
            jax.ShapeDtypeStruct((_NPAD, d), f32),
            jax.ShapeDtypeStruct((_NPAD, 16), f32),
        ],
        scratch_types=[
            pltpu.VMEM((_NPC, d), f32),
            pltpu.VMEM((_NPC, d), f32),
            pltpu.VMEM((_NPC, d), f32),
            pltpu.VMEM((_NPC, d), f32),
            pltpu.VMEM((_NPC, 16), f32),
            pltpu.VMEM((_T,), jnp.int32),
            pltpu.VMEM((_T,), jnp.int32),
            pltpu.VMEM((_T + 16,), jnp.int32),
            pltpu.VMEM((_T + 16,), jnp.int32),
            pltpu.VMEM((_GB,), jnp.int32),
            pltpu.VMEM((_GB, d), f32),
            pltpu.VMEM((_GB, d), f32),
            pltpu.SemaphoreType.DMA,
            pltpu.SemaphoreType.DMA,
            pltpu.SemaphoreType.DMA,
            pltpu.SemaphoreType.DMA,
        ],
        compiler_params=sc_params,
    )
    s1, s2, mx, mn, deg = seg_kernel(src, dst, a, c)
    s1, s2, mx, mn, deg = (s1[:n], s2[:n], mx[:n], mn[:n], deg[:n])

    out0, bns, bnq = pl.pallas_call(
        functools.partial(_post_body, d=d),
        grid=(nb_n,),
        in_specs=[
            pl.BlockSpec((blk_n, d), lambda i: (i, 0)),
            pl.BlockSpec((blk_n, d), lambda i: (i, 0)),
            pl.BlockSpec((blk_n, d), lambda i: (i, 0)),
            pl.BlockSpec((blk_n, d), lambda i: (i, 0)),
            pl.BlockSpec((blk_n, d), lambda i: (i, 0)),
            pl.BlockSpec((blk_n, d), lambda i: (i, 0)),
            pl.BlockSpec((blk_n, 16), lambda i: (i, 0)),
            pl.BlockSpec((blk_n, 1), lambda i: (i, 0)),
            pl.BlockSpec((13 * d, d), lambda i: (0, 0)),
            pl.BlockSpec((1, d), lambda i: (0, 0)),
        ],
        out_specs=[
            pl.BlockSpec((blk_n, d), lambda i: (i, 0)),
            pl.BlockSpec((1, d), lambda i: (0, 0)),
            pl.BlockSpec((1, d), lambda i: (0, 0)),
        ],
        out_shape=[
            jax.ShapeDtypeStruct((n, d), jnp.float32),
            jax.ShapeDtypeStruct((1, d), jnp.float32),
            jax.ShapeDtypeStruct((1, d), jnp.float32),
        ],
    )(h, b, s1, s2, mx, mn, deg, snorm_n, W_post, b_post2)

    out = pl.pallas_call(
        functools.partial(_bn_body, n=float(n)),
        grid=(nb_n,),
        in_specs=[
            pl.BlockSpec((blk_n, d), lambda i: (i, 0)),
            pl.BlockSpec((1, d), lambda i: (0, 0)),
            pl.BlockSpec((1, d), lambda i: (0, 0)),
            pl.BlockSpec((1, d), lambda i: (0, 0)),
            pl.BlockSpec((1, d), lambda i: (0, 0)),
        ],
        out_specs=pl.BlockSpec((blk_n, d), lambda i: (i, 0)),
        out_shape=jax.ShapeDtypeStruct((n, d), jnp.float32),
    )(out0, bns, bnq, gamma2, beta2)
    return out
